# Initial kernel scaffold; baseline (speedup 1.0000x reference)
import jax
import jax.numpy as jnp
from jax.experimental import pallas as pl


def kernel(school_idx, goal_idx, method_idx, subject_multi_hot, grade_multi_hot, school_emb, goal_emb, method_emb, W_sub, b_sub, W_gr, b_gr, W1, b1, W2, b2, W3, b3):
    raise NotImplementedError("write your pallas kernel here")


# speedup vs baseline: 5.8726x; 5.8726x over previous
import functools

import jax
import jax.numpy as jnp
from jax import lax
from jax.experimental import pallas as pl

TB = 2048


def _tower_body(si_ref, gi_ref, mi_ref, subM_ref, grM_ref,
                se_ref, ge_ref, me_ref, Wsub_ref, bsub_ref, Wgr_ref, bgr_ref,
                W1_ref, b1_ref, W2_ref, b2_ref, W3_ref, b3_ref, out_ref):
    f32 = jnp.float32
    W1 = W1_ref[...]
    Ts = jnp.dot(se_ref[...], W1[0:32, :], preferred_element_type=f32)
    Tg = jnp.dot(ge_ref[...], W1[32:64, :], preferred_element_type=f32)
    Tm = jnp.dot(me_ref[...], W1[64:96, :], preferred_element_type=f32)
    Wsub1 = jnp.dot(Wsub_ref[...], W1[96:128, :], preferred_element_type=f32)
    Wgr1 = jnp.dot(Wgr_ref[...], W1[128:160, :], preferred_element_type=f32)
    bias1 = (b1_ref[...]
             + jnp.dot(bsub_ref[...], W1[96:128, :], preferred_element_type=f32)
             + jnp.dot(bgr_ref[...], W1[128:160, :], preferred_element_type=f32))

    s_idx = si_ref[0, 0, :]
    g_idx = gi_ref[0, 0, :]
    m_idx = mi_ref[0, 0, :]
    oh_s = (s_idx[:, None] == lax.broadcasted_iota(jnp.int32, (TB, 102), 1)).astype(f32)
    oh_g = (g_idx[:, None] == lax.broadcasted_iota(jnp.int32, (TB, 22), 1)).astype(f32)
    oh_m = (m_idx[:, None] == lax.broadcasted_iota(jnp.int32, (TB, 12), 1)).astype(f32)

    h1 = (jnp.dot(oh_s, Ts, preferred_element_type=f32)
          + jnp.dot(oh_g, Tg, preferred_element_type=f32)
          + jnp.dot(oh_m, Tm, preferred_element_type=f32)
          + jnp.dot(subM_ref[...], Wsub1, preferred_element_type=f32)
          + jnp.dot(grM_ref[...], Wgr1, preferred_element_type=f32)
          + bias1)
    h1 = jnp.maximum(h1, 0.0)
    h2 = jnp.maximum(jnp.dot(h1, W2_ref[...], preferred_element_type=f32) + b2_ref[...], 0.0)
    out_ref[...] = jnp.dot(h2, W3_ref[...], preferred_element_type=f32) + b3_ref[...]


def kernel(school_idx, goal_idx, method_idx, subject_multi_hot, grade_multi_hot,
           school_emb, goal_emb, method_emb, W_sub, b_sub, W_gr, b_gr,
           W1, b1, W2, b2, W3, b3):
    B = school_idx.shape[0]
    nb = B // TB
    si = school_idx.astype(jnp.int32).reshape(nb, 1, TB)
    gi = goal_idx.astype(jnp.int32).reshape(nb, 1, TB)
    mi = method_idx.astype(jnp.int32).reshape(nb, 1, TB)

    def idx_spec():
        return pl.BlockSpec((1, 1, TB), lambda i: (i, 0, 0))

    def batch_spec(w):
        return pl.BlockSpec((TB, w), lambda i: (i, 0))

    def full_spec(shape):
        return pl.BlockSpec(shape, lambda i: (0,) * len(shape))

    out = pl.pallas_call(
        _tower_body,
        grid=(nb,),
        in_specs=[
            idx_spec(), idx_spec(), idx_spec(),
            batch_spec(15), batch_spec(12),
            full_spec((102, 32)), full_spec((22, 32)), full_spec((12, 32)),
            full_spec((15, 32)), full_spec((1, 32)),
            full_spec((12, 32)), full_spec((1, 32)),
            full_spec((160, 128)), full_spec((1, 128)),
            full_spec((128, 64)), full_spec((1, 64)),
            full_spec((64, 32)), full_spec((1, 32)),
        ],
        out_specs=pl.BlockSpec((TB, 32), lambda i: (i, 0)),
        out_shape=jax.ShapeDtypeStruct((B, 32), jnp.float32),
    )(si, gi, mi, subject_multi_hot, grade_multi_hot,
      school_emb, goal_emb, method_emb,
      W_sub, b_sub.reshape(1, 32), W_gr, b_gr.reshape(1, 32),
      W1, b1.reshape(1, 128), W2, b2.reshape(1, 64), W3, b3.reshape(1, 32))
    return out
